# Initial kernel scaffold; baseline (speedup 1.0000x reference)
import jax
import jax.numpy as jnp
from jax import lax
from jax.experimental import pallas as pl
from jax.experimental.pallas import tpu as pltpu
from jax.experimental.pallas import tpu_sc as plsc

N = 10000
E = 160000
D_IN = 128
DM = 64
DH = 128
DE = 16
DXP = 16
LAYERS = 4

_NC, _NS = 2, 16
_NW = _NC * _NS
_C = 128
_NCHUNK = E // _C
_ITERS = (_NCHUNK + _NW - 1) // _NW

_BE = 1000
_BN = 2000

f32 = jnp.float32


def _sc_gather_body(A_h, B_h, xp_h, xn_h, src_h, dst_h, g_h, xd_h,
                    sidx, didx, g_v, xd_v, sem):
    c = lax.axis_index("c")
    s = lax.axis_index("s")
    w = s * _NC + c

    def step(i, carry):
        j = i * _NW + w

        @pl.when(j < _NCHUNK)
        def _():
            base = pl.multiple_of(j * _C, _C)
            pltpu.sync_copy(src_h.at[pl.ds(base, _C)], sidx)
            pltpu.sync_copy(dst_h.at[pl.ds(base, _C)], didx)
            pltpu.async_copy(A_h.at[sidx], g_v, sem).wait()
            pltpu.async_copy(B_h.at[didx], g_v, sem, add=True).wait()
            pltpu.async_copy(xp_h.at[didx], xd_v, sem).wait()
            pltpu.async_copy(xn_h.at[sidx], xd_v, sem, add=True).wait()
            pltpu.sync_copy(g_v, g_h.at[pl.ds(base, _C)])
            pltpu.sync_copy(xd_v, xd_h.at[pl.ds(base, _C)])
        return carry

    lax.fori_loop(0, _ITERS, step, None)


def _sc_scatter_body(m_h, v_h, dst_h, zM_h, zX_h, Mp_h, Xp_h,
                     didx, m_v, v_v, Macc, Xacc, sem):
    c = lax.axis_index("c")
    s = lax.axis_index("s")
    w = s * _NC + c

    @pl.when(s == 0)
    def _():
        pltpu.sync_copy(zM_h, Macc)
        pltpu.sync_copy(zX_h, Xacc)
    plsc.subcore_barrier()

    def step(i, carry):
        j = i * _NW + w

        @pl.when(j < _NCHUNK)
        def _():
            base = pl.multiple_of(j * _C, _C)
            pltpu.sync_copy(dst_h.at[pl.ds(base, _C)], didx)
            pltpu.sync_copy(m_h.at[pl.ds(base, _C)], m_v)
            pltpu.sync_copy(v_h.at[pl.ds(base, _C)], v_v)
            pltpu.sync_copy(m_v, Macc.at[didx], add=True)
            pltpu.sync_copy(v_v, Xacc.at[didx], add=True)
        return carry

    lax.fori_loop(0, _ITERS, step, None)
    plsc.subcore_barrier()

    @pl.when(s == 0)
    def _():
        pltpu.sync_copy(Macc, Mp_h.at[c])
        pltpu.sync_copy(Xacc, Xp_h.at[c])


_sc_mesh = plsc.VectorSubcoreMesh(core_axis_name="c", subcore_axis_name="s")

_sc_gather = pl.kernel(
    _sc_gather_body,
    out_type=[jax.ShapeDtypeStruct((E, DH), f32),
              jax.ShapeDtypeStruct((E, DXP), f32)],
    mesh=_sc_mesh,
    scratch_types=[pltpu.VMEM((_C,), jnp.int32),
                   pltpu.VMEM((_C,), jnp.int32),
                   pltpu.VMEM((_C, DH), f32),
                   pltpu.VMEM((_C, DXP), f32),
                   pltpu.SemaphoreType.DMA],
)

_sc_scatter = pl.kernel(
    _sc_scatter_body,
    out_type=[jax.ShapeDtypeStruct((_NC, N, DM), f32),
              jax.ShapeDtypeStruct((_NC, N, DXP), f32)],
    mesh=_sc_mesh,
    scratch_types=[pltpu.VMEM((_C,), jnp.int32),
                   pltpu.VMEM((_C, DM), f32),
                   pltpu.VMEM((_C, DXP), f32),
                   pltpu.VMEM_SHARED((N, DM), f32),
                   pltpu.VMEM_SHARED((N, DXP), f32),
                   pltpu.SemaphoreType.DMA],
)


def _dot(a, b):
    return jnp.dot(a, b, preferred_element_type=f32)


def _prologue_body(nf, pW, pb, W1a, W1b, xp, h_o, A_o, B_o, xn_o):
    h = jnp.maximum(_dot(nf[...], pW[...]) + pb[...], 0.0)
    h_o[...] = h
    A_o[...] = _dot(h, W1a[...])
    B_o[...] = _dot(h, W1b[...])
    xn_o[...] = -xp[...]


def _edge_body(g, xd, ef, W1c, M3, b1, mW2, mb2, xW1, xb1, xW2t, xbrow,
               m_o, vec_o):
    xdv = xd[...]
    diff = xdv + 1e-6
    t = jnp.maximum(
        g[...] + _dot(ef[...], W1c[...]) + _dot(diff * diff, M3[...])
        + b1[...], 0.0)
    m = _dot(t, mW2[...]) + mb2[...]
    t2 = jnp.maximum(_dot(m, xW1[...]) + xb1[...], 0.0)
    mx = _dot(t2, xW2t[...]) + xbrow[...]
    m_o[...] = m
    vec_o[...] = -xdv * mx


def _node_body(h, Mp, xdp, xp, W1h, W1M, nb1, nW2, nb2, W1a, W1b,
               hn_o, A_o, B_o, xpn_o, xnn_o):
    M = Mp[0] + Mp[1]
    z = jnp.maximum(_dot(h[...], W1h[...]) + _dot(M, W1M[...]) + nb1[...],
                    0.0)
    hn = _dot(z, nW2[...]) + nb2[...]
    xn = xp[...] + xdp[0] + xdp[1]
    hn_o[...] = hn
    A_o[...] = _dot(hn, W1a[...])
    B_o[...] = _dot(hn, W1b[...])
    xpn_o[...] = xn
    xnn_o[...] = -xn


def _readout_body(h, ro1, rb1, ro2, rb2, tW, tb, y_o):
    r = jnp.maximum(_dot(h[...], ro1[...]) + rb1[...], 0.0)
    r = _dot(r, ro2[...]) + rb2[...]
    srow = jnp.sum(r, axis=0, keepdims=True)
    y_o[...] = _dot(srow, tW[...]) + tb[...]


def _full(shape):
    return pl.BlockSpec(shape, lambda *_: tuple(0 for _ in shape))


_prologue = pl.pallas_call(
    _prologue_body,
    grid=(N // _BN,),
    in_specs=[pl.BlockSpec((_BN, D_IN), lambda i: (i, 0)),
              _full((D_IN, DM)), _full((1, DM)),
              _full((DM, DH)), _full((DM, DH)),
              pl.BlockSpec((_BN, DXP), lambda i: (i, 0))],
    out_specs=[pl.BlockSpec((_BN, DM), lambda i: (i, 0)),
               pl.BlockSpec((_BN, DH), lambda i: (i, 0)),
               pl.BlockSpec((_BN, DH), lambda i: (i, 0)),
               pl.BlockSpec((_BN, DXP), lambda i: (i, 0))],
    out_shape=[jax.ShapeDtypeStruct((N, DM), f32),
               jax.ShapeDtypeStruct((N, DH), f32),
               jax.ShapeDtypeStruct((N, DH), f32),
               jax.ShapeDtypeStruct((N, DXP), f32)],
)

_edge = pl.pallas_call(
    _edge_body,
    grid=(E // _BE,),
    in_specs=[pl.BlockSpec((_BE, DH), lambda i: (i, 0)),
              pl.BlockSpec((_BE, DXP), lambda i: (i, 0)),
              pl.BlockSpec((_BE, DE), lambda i: (i, 0)),
              _full((DE, DH)), _full((DXP, DH)), _full((1, DH)),
              _full((DH, DM)), _full((1, DM)),
              _full((DM, DM)), _full((1, DM)),
              _full((DM, DXP)), _full((1, DXP))],
    out_specs=[pl.BlockSpec((_BE, DM), lambda i: (i, 0)),
               pl.BlockSpec((_BE, DXP), lambda i: (i, 0))],
    out_shape=[jax.ShapeDtypeStruct((E, DM), f32),
               jax.ShapeDtypeStruct((E, DXP), f32)],
)

_node = pl.pallas_call(
    _node_body,
    grid=(N // _BN,),
    in_specs=[pl.BlockSpec((_BN, DM), lambda i: (i, 0)),
              pl.BlockSpec((_NC, _BN, DM), lambda i: (0, i, 0)),
              pl.BlockSpec((_NC, _BN, DXP), lambda i: (0, i, 0)),
              pl.BlockSpec((_BN, DXP), lambda i: (i, 0)),
              _full((DM, DH)), _full((DM, DH)), _full((1, DH)),
              _full((DH, DM)), _full((1, DM)),
              _full((DM, DH)), _full((DM, DH))],
    out_specs=[pl.BlockSpec((_BN, DM), lambda i: (i, 0)),
               pl.BlockSpec((_BN, DH), lambda i: (i, 0)),
               pl.BlockSpec((_BN, DH), lambda i: (i, 0)),
               pl.BlockSpec((_BN, DXP), lambda i: (i, 0)),
               pl.BlockSpec((_BN, DXP), lambda i: (i, 0))],
    out_shape=[jax.ShapeDtypeStruct((N, DM), f32),
               jax.ShapeDtypeStruct((N, DH), f32),
               jax.ShapeDtypeStruct((N, DH), f32),
               jax.ShapeDtypeStruct((N, DXP), f32),
               jax.ShapeDtypeStruct((N, DXP), f32)],
)

_readout = pl.pallas_call(
    _readout_body,
    in_specs=[_full((N, DM)),
              _full((DM, DM)), _full((1, DM)),
              _full((DM, DM)), _full((1, DM)),
              _full((DM, DH)), _full((1, DH))],
    out_specs=_full((1, DH)),
    out_shape=jax.ShapeDtypeStruct((1, DH), f32),
)


def kernel(node_feats, edge_feats, x, params, edge_index):
    p = params
    src = edge_index[0]
    dst = edge_index[1]

    W1a = p['m_W1'][0:DM]
    W1b = p['m_W1'][DM:2 * DM]
    W1c = p['m_W1'][2 * DM:2 * DM + DE]
    w1d = p['m_W1'][2 * DM + DE]
    mask3 = (jnp.arange(DXP) < 3).astype(f32)
    M3 = mask3[:, None] * w1d[None, :]
    xbrow = (jnp.zeros((DXP,), f32).at[:3].set(p['x_bias'])
             + p['x_b2']).reshape(1, DXP)
    xW2t = jnp.tile(p['x_W2'], (1, DXP))
    mb2 = (p['m_b2'] + p['m_bias']).reshape(1, DM)
    nb2 = (p['nm_b2'] + p['node_bias']).reshape(1, DM)
    W1h = p['nm_W1'][:DM]
    W1M = p['nm_W1'][DM:]
    xpad0 = jnp.pad(x, ((0, 0), (0, DXP - 3)))
    tWpad = jnp.pad(p['task_W'], ((0, 0), (0, DH - 1)))
    tbpad = jnp.pad(p['task_b'], (0, DH - 1)).reshape(1, DH)
    zM = jnp.zeros((N, DM), f32)
    zX = jnp.zeros((N, DXP), f32)

    h, A, B, xneg = _prologue(node_feats, p['proj_W'],
                              p['proj_b'].reshape(1, DM), W1a, W1b, xpad0)
    xpad = xpad0
    for _ in range(LAYERS):
        g, xd = _sc_gather(A, B, xpad, xneg, src, dst)
        m, vec = _edge(g, xd, edge_feats, W1c, M3,
                       p['m_b1'].reshape(1, DH), p['m_W2'], mb2,
                       p['x_W1'], p['x_b1'].reshape(1, DM), xW2t, xbrow)
        Mp, Xp = _sc_scatter(m, vec, dst, zM, zX)
        h, A, B, xpad, xneg = _node(h, Mp, Xp, xpad, W1h, W1M,
                                    p['nm_b1'].reshape(1, DH), p['nm_W2'],
                                    nb2, W1a, W1b)

    ypad = _readout(h, p['ro_W1'], p['ro_b1'].reshape(1, DM),
                    p['ro_W2'], p['ro_b2'].reshape(1, DM), tWpad, tbpad)
    return ypad[:, :1]


# speedup vs baseline: 2.9657x; 2.9657x over previous
import functools

import jax
import jax.numpy as jnp
from jax import lax
from jax.experimental import pallas as pl
from jax.experimental.pallas import tpu as pltpu
from jax.experimental.pallas import tpu_sc as plsc

N = 10000
E = 160000
D_IN = 128
DM = 64
DH = 128
DE = 16
DXP = 16
DMV = 128
DK = 152
LAYERS = 4

_NC, _NS = 2, 16
_NW = _NC * _NS
_C = 128
_NCHUNK = E // _C
_ITERS = (_NCHUNK + _NW - 1) // _NW

_BE = 1000
_BN = 2000

f32 = jnp.float32
bf16 = jnp.bfloat16


def _sc_gather_body(h_h, xp_h, xn_h, src_h, dst_h, hh_h, xd_h,
                    sidx, didx, bufj, bufi, xbuf, sem):
    c = lax.axis_index("c")
    s = lax.axis_index("s")
    w = s * _NC + c

    def step(i, carry):
        j = i * _NW + w

        @pl.when(j < _NCHUNK)
        def _():
            base = pl.multiple_of(j * _C, _C)
            pltpu.sync_copy(src_h.at[pl.ds(base, _C)], sidx)
            pltpu.sync_copy(dst_h.at[pl.ds(base, _C)], didx)
            pltpu.async_copy(h_h.at[sidx], bufj, sem).wait()
            pltpu.async_copy(h_h.at[didx], bufi, sem).wait()
            pltpu.async_copy(xp_h.at[didx], xbuf, sem).wait()
            pltpu.async_copy(xn_h.at[sidx], xbuf, sem, add=True).wait()
            pltpu.sync_copy(bufj, hh_h.at[pl.ds(base, _C), pl.ds(0, DM)])
            pltpu.sync_copy(bufi, hh_h.at[pl.ds(base, _C), pl.ds(DM, DM)])
            pltpu.sync_copy(xbuf, xd_h.at[pl.ds(base, _C)])
        return carry

    lax.fori_loop(0, _ITERS, step, None)


def _sc_scatter_body(mv_h, dst_h, z_h, Mp_h, didx, buf, acc, sem):
    c = lax.axis_index("c")
    s = lax.axis_index("s")
    w = s * _NC + c

    @pl.when(s == 0)
    def _():
        pltpu.sync_copy(z_h, acc)
    plsc.subcore_barrier()

    def step(i, carry):
        j = i * _NW + w

        @pl.when(j < _NCHUNK)
        def _():
            base = pl.multiple_of(j * _C, _C)
            pltpu.sync_copy(dst_h.at[pl.ds(base, _C)], didx)
            pltpu.sync_copy(mv_h.at[pl.ds(base, _C)], buf)
            pltpu.sync_copy(buf, acc.at[didx], add=True)
        return carry

    lax.fori_loop(0, _ITERS, step, None)
    plsc.subcore_barrier()

    @pl.when(s == 0)
    def _():
        pltpu.sync_copy(acc, Mp_h.at[c])


@functools.lru_cache(maxsize=1)
def _get_sc_kernels():
    mesh = plsc.VectorSubcoreMesh(core_axis_name="c", subcore_axis_name="s",
                                  num_cores=_NC, num_subcores=_NS)
    gather = pl.kernel(
        _sc_gather_body,
        out_type=[jax.ShapeDtypeStruct((E, 2 * DM), f32),
                  jax.ShapeDtypeStruct((E, DXP), f32)],
        mesh=mesh,
        scratch_types=[pltpu.VMEM((_C,), jnp.int32),
                       pltpu.VMEM((_C,), jnp.int32),
                       pltpu.VMEM((_C, DM), f32),
                       pltpu.VMEM((_C, DM), f32),
                       pltpu.VMEM((_C, DXP), f32),
                       pltpu.SemaphoreType.DMA],
        compiler_params=pltpu.CompilerParams(use_tc_tiling_on_sc=False),
    )
    scatter = pl.kernel(
        _sc_scatter_body,
        out_type=jax.ShapeDtypeStruct((_NC, N, DMV), f32),
        mesh=mesh,
        scratch_types=[pltpu.VMEM((_C,), jnp.int32),
                       pltpu.VMEM((_C, DMV), f32),
                       pltpu.VMEM_SHARED((N, DMV), f32),
                       pltpu.SemaphoreType.DMA],
    )
    return gather, scatter


def _dot(a, b):
    return jnp.dot(a.astype(bf16), b.astype(bf16),
                   preferred_element_type=f32)


def _prologue_body(nf, pW, pb, xp, h_o, xn_o):
    h_o[...] = jnp.maximum(_dot(nf[...], pW[...]) + pb[...], 0.0)
    xn_o[...] = -xp[...]


def _edge_body(hh, xd, ef, W1p, mask3, b1, mW2, mb2, xW1, xb1, xW2t,
               xbrow, mv_o):
    xdv = xd[...]
    diff = xdv + 1e-6
    d2 = jnp.sum(diff * diff * mask3[...], axis=1, keepdims=True)
    nrow = hh.shape[0]
    z = jnp.zeros((nrow, DK - 2 * DM - DE - 1), f32)
    cat = jnp.concatenate([hh[...], ef[...], d2, z], axis=1)
    t = jnp.maximum(_dot(cat, W1p[...]) + b1[...], 0.0)
    m = _dot(t, mW2[...]) + mb2[...]
    t2 = jnp.maximum(_dot(m, xW1[...]) + xb1[...], 0.0)
    mx = _dot(t2, xW2t[...]) + xbrow[...]
    vec = -xdv * mx
    zv = jnp.zeros((nrow, DMV - DM - DXP), f32)
    mv_o[...] = jnp.concatenate([m, vec, zv], axis=1)


def _node_body(h, Mp, xp, nW1, nb1, nW2, nb2, hn_o, xpn_o, xnn_o):
    acc = Mp[0] + Mp[1]
    M = acc[:, :DM]
    xdel = acc[:, DM:DM + DXP]
    hcat = jnp.concatenate([h[...], M], axis=1)
    z = jnp.maximum(_dot(hcat, nW1[...]) + nb1[...], 0.0)
    hn = _dot(z, nW2[...]) + nb2[...]
    xn = xp[...] + xdel
    hn_o[...] = hn
    xpn_o[...] = xn
    xnn_o[...] = -xn


def _readout_body(h, ro1, rb1, ro2, rb2, tW, tb, y_o):
    r = jnp.maximum(_dot(h[...], ro1[...]) + rb1[...], 0.0)
    r = _dot(r, ro2[...]) + rb2[...]
    srow = jnp.sum(r, axis=0, keepdims=True)
    y_o[...] = _dot(srow, tW[...]) + tb[...]


def _full(shape):
    return pl.BlockSpec(shape, lambda *_: tuple(0 for _ in shape))


_prologue = pl.pallas_call(
    _prologue_body,
    grid=(N // _BN,),
    in_specs=[pl.BlockSpec((_BN, D_IN), lambda i: (i, 0)),
              _full((D_IN, DM)), _full((1, DM)),
              pl.BlockSpec((_BN, DXP), lambda i: (i, 0))],
    out_specs=[pl.BlockSpec((_BN, DM), lambda i: (i, 0)),
               pl.BlockSpec((_BN, DXP), lambda i: (i, 0))],
    out_shape=[jax.ShapeDtypeStruct((N, DM), f32),
               jax.ShapeDtypeStruct((N, DXP), f32)],
)

_edge = pl.pallas_call(
    _edge_body,
    grid=(E // _BE,),
    in_specs=[pl.BlockSpec((_BE, 2 * DM), lambda i: (i, 0)),
              pl.BlockSpec((_BE, DXP), lambda i: (i, 0)),
              pl.BlockSpec((_BE, DE), lambda i: (i, 0)),
              _full((DK, DH)), _full((1, DXP)), _full((1, DH)),
              _full((DH, DM)), _full((1, DM)),
              _full((DM, DM)), _full((1, DM)),
              _full((DM, DXP)), _full((1, DXP))],
    out_specs=pl.BlockSpec((_BE, DMV), lambda i: (i, 0)),
    out_shape=jax.ShapeDtypeStruct((E, DMV), f32),
)

_node = pl.pallas_call(
    _node_body,
    grid=(N // _BN,),
    in_specs=[pl.BlockSpec((_BN, DM), lambda i: (i, 0)),
              pl.BlockSpec((_NC, _BN, DMV), lambda i: (0, i, 0)),
              pl.BlockSpec((_BN, DXP), lambda i: (i, 0)),
              _full((DH, DH)), _full((1, DH)),
              _full((DH, DM)), _full((1, DM))],
    out_specs=[pl.BlockSpec((_BN, DM), lambda i: (i, 0)),
               pl.BlockSpec((_BN, DXP), lambda i: (i, 0)),
               pl.BlockSpec((_BN, DXP), lambda i: (i, 0))],
    out_shape=[jax.ShapeDtypeStruct((N, DM), f32),
               jax.ShapeDtypeStruct((N, DXP), f32),
               jax.ShapeDtypeStruct((N, DXP), f32)],
)

_readout = pl.pallas_call(
    _readout_body,
    in_specs=[_full((N, DM)),
              _full((DM, DM)), _full((1, DM)),
              _full((DM, DM)), _full((1, DM)),
              _full((DM, DH)), _full((1, DH))],
    out_specs=_full((1, DH)),
    out_shape=jax.ShapeDtypeStruct((1, DH), f32),
)


def kernel(node_feats, edge_feats, x, params, edge_index):
    p = params
    src = edge_index[0]
    dst = edge_index[1]

    W1p = jnp.pad(p['m_W1'], ((0, DK - 2 * DM - DE - 1), (0, 0)))
    mask3 = (jnp.arange(DXP) < 3).astype(f32).reshape(1, DXP)
    xbrow = (jnp.zeros((DXP,), f32).at[:3].set(p['x_bias'])
             + p['x_b2']).reshape(1, DXP)
    xW2t = jnp.tile(p['x_W2'], (1, DXP))
    mb2 = (p['m_b2'] + p['m_bias']).reshape(1, DM)
    nb2 = (p['nm_b2'] + p['node_bias']).reshape(1, DM)
    xpad0 = jnp.pad(x, ((0, 0), (0, DXP - 3)))
    tWpad = jnp.pad(p['task_W'], ((0, 0), (0, DH - 1)))
    tbpad = jnp.pad(p['task_b'], (0, DH - 1)).reshape(1, DH)
    zacc = jnp.zeros((N, DMV), f32)

    h, xneg = _prologue(node_feats, p['proj_W'],
                        p['proj_b'].reshape(1, DM), xpad0)
    xpad = xpad0
    sc_gather, sc_scatter = _get_sc_kernels()
    for _ in range(LAYERS):
        hh, xd = sc_gather(h, xpad, xneg, src, dst)
        mv = _edge(hh, xd, edge_feats, W1p, mask3,
                   p['m_b1'].reshape(1, DH), p['m_W2'], mb2,
                   p['x_W1'], p['x_b1'].reshape(1, DM), xW2t, xbrow)
        Mp = sc_scatter(mv, dst, zacc)
        h, xpad, xneg = _node(h, Mp, xpad, p['nm_W1'],
                              p['nm_b1'].reshape(1, DH), p['nm_W2'], nb2)

    ypad = _readout(h, p['ro_W1'], p['ro_b1'].reshape(1, DM),
                    p['ro_W2'], p['ro_b2'].reshape(1, DM), tWpad, tbpad)
    return ypad[:, :1]
